# baseline (speedup 1.0000x reference)
import functools

import jax
import jax.numpy as jnp
from jax import lax
from jax.experimental import pallas as pl
from jax.experimental.pallas import tpu as pltpu
from jax.experimental.pallas import tpu_sc as plsc

S, B, D = 8192, 4, 768
NC, NS, L = 2, 16, 16
NW = NC * NS
S_PER_W = S // NW
CH = 4
NG = D // L

_mesh = plsc.VectorSubcoreMesh(core_axis_name="c", subcore_axis_name="s")


def _sc_body(x_hbm, ids_hbm, tpe_hbm, trk_hbm, out_hbm,
             x_v, t_v, trk_rows, ids_v, sem):
    wid = lax.axis_index("s") * NC + lax.axis_index("c")
    s0 = wid * S_PER_W
    pltpu.sync_copy(ids_hbm.at[pl.ds(s0 * B, S_PER_W * B)], ids_v)

    def chunk(ci, _):
        sb = s0 + ci * CH
        pltpu.sync_copy(x_hbm.at[pl.ds(sb, CH)], x_v)
        pltpu.sync_copy(tpe_hbm.at[pl.ds(sb, CH)], t_v)
        idx = ids_v.at[pl.ds(ci * (CH * B), CH * B)]
        pltpu.async_copy(trk_hbm.at[idx], trk_rows, sem).wait()
        for si in range(CH):
            for bi in range(B):
                for g in range(NG):
                    sl = pl.ds(g * L, L)
                    x_v[si, bi, sl] = (
                        x_v[si, bi, sl] + t_v[si, sl] + trk_rows[si * B + bi, sl]
                    )
        pltpu.sync_copy(x_v, out_hbm.at[pl.ds(sb, CH)])
        return ()

    lax.fori_loop(0, S_PER_W // CH, chunk, ())


@functools.partial(
    pl.kernel,
    out_type=jax.ShapeDtypeStruct((S, B, D), jnp.float32),
    mesh=_mesh,
    scratch_types=[
        pltpu.VMEM((CH, B, D), jnp.float32),
        pltpu.VMEM((CH, D), jnp.float32),
        pltpu.VMEM((CH * B, D), jnp.float32),
        pltpu.VMEM((S_PER_W * B,), jnp.int32),
        pltpu.SemaphoreType.DMA,
    ],
)
def _sc_kernel(x_hbm, ids_hbm, tpe_hbm, trk_hbm, out_hbm,
               x_v, t_v, trk_rows, ids_v, sem):
    _sc_body(x_hbm, ids_hbm, tpe_hbm, trk_hbm, out_hbm,
             x_v, t_v, trk_rows, ids_v, sem)


@jax.jit
def kernel(x, track_ids, time_pe, track_pe):
    ids_flat = track_ids.reshape(S * B)
    return _sc_kernel(x, ids_flat, time_pe[:S], track_pe)


# speedup vs baseline: 4.3576x; 4.3576x over previous
import functools

import jax
import jax.numpy as jnp
from jax import lax
from jax.experimental import pallas as pl
from jax.experimental.pallas import tpu as pltpu
from jax.experimental.pallas import tpu_sc as plsc

S, B, D = 8192, 4, 768
NC, NS, L = 2, 16, 16
NW = NC * NS
S_SC = 1024
S_PER_W = S_SC // NW
CH = 4
NG = D // L
S_BLK = 256
TC_OFF = S_SC // S_BLK

_mesh = plsc.VectorSubcoreMesh(core_axis_name="c", subcore_axis_name="s")


def _sc_body(x_hbm, ids_hbm, tpe_hbm, trk_hbm, out_hbm,
             x_v, t_v, trk_rows, ids_v, sem):
    wid = lax.axis_index("s") * NC + lax.axis_index("c")
    s0 = wid * S_PER_W
    pltpu.sync_copy(ids_hbm.at[pl.ds(s0 * B, S_PER_W * B)], ids_v)

    def chunk(ci, _):
        sb = s0 + ci * CH
        pltpu.sync_copy(x_hbm.at[pl.ds(sb, CH)], x_v)
        pltpu.sync_copy(tpe_hbm.at[pl.ds(sb, CH)], t_v)
        idx = ids_v.at[pl.ds(ci * (CH * B), CH * B)]
        pltpu.async_copy(trk_hbm.at[idx], trk_rows, sem).wait()
        for si in range(CH):
            for bi in range(B):
                for g in range(NG):
                    sl = pl.ds(g * L, L)
                    x_v[si, bi, sl] = (
                        x_v[si, bi, sl] + t_v[si, sl] + trk_rows[si * B + bi, sl]
                    )
        pltpu.sync_copy(x_v, out_hbm.at[pl.ds(sb, CH)])
        return ()

    lax.fori_loop(0, S_PER_W // CH, chunk, ())


@functools.partial(
    pl.kernel,
    out_type=jax.ShapeDtypeStruct((S_SC, B, D), jnp.float32),
    mesh=_mesh,
    scratch_types=[
        pltpu.VMEM((CH, B, D), jnp.float32),
        pltpu.VMEM((CH, D), jnp.float32),
        pltpu.VMEM((CH * B, D), jnp.float32),
        pltpu.VMEM((S_PER_W * B,), jnp.int32),
        pltpu.SemaphoreType.DMA,
    ],
)
def _sc_kernel(x_hbm, ids_hbm, tpe_hbm, trk_hbm, out_hbm,
               x_v, t_v, trk_rows, ids_v, sem):
    _sc_body(x_hbm, ids_hbm, tpe_hbm, trk_hbm, out_hbm,
             x_v, t_v, trk_rows, ids_v, sem)


def _tc_body(x_ref, ids_ref, tpe_ref, trk_ref, o_ref):
    x = x_ref[...]
    ids = ids_ref[...]
    t = tpe_ref[...]
    acc = x + t[:, None, :]
    enc = jnp.broadcast_to(trk_ref[0:1, :][None, :, :], x.shape)
    for k in range(1, 8):
        enc = jnp.where(ids == k, trk_ref[k:k + 1, :][None, :, :], enc)
    o_ref[...] = acc + enc


def _tc_kernel(x, ids3, tpe, trk):
    grid = ((S - S_SC) // S_BLK,)
    return pl.pallas_call(
        _tc_body,
        grid=grid,
        in_specs=[
            pl.BlockSpec((S_BLK, B, D), lambda i: (TC_OFF + i, 0, 0)),
            pl.BlockSpec((S_BLK, B, 1), lambda i: (TC_OFF + i, 0, 0)),
            pl.BlockSpec((S_BLK, D), lambda i: (TC_OFF + i, 0)),
            pl.BlockSpec((8, D), lambda i: (0, 0)),
        ],
        out_specs=pl.BlockSpec((S_BLK, B, D), lambda i: (TC_OFF + i, 0, 0)),
        out_shape=jax.ShapeDtypeStruct((S, B, D), jnp.float32),
        compiler_params=pltpu.CompilerParams(
            dimension_semantics=("arbitrary",),
        ),
    )(x, ids3, tpe, trk)


@jax.jit
def kernel(x, track_ids, time_pe, track_pe):
    ids_flat = track_ids.reshape(S * B)
    out_sc = _sc_kernel(x, ids_flat, time_pe[:S], track_pe)
    ids3 = track_ids.reshape(S, B, 1)
    out_tc = _tc_kernel(x, ids3, time_pe[:S], track_pe)
    return lax.dynamic_update_slice(out_tc, out_sc, (0, 0, 0))
